# baseline (speedup 1.0000x reference)
import functools

import jax
import jax.numpy as jnp
import numpy as np
from jax import lax
from jax.experimental import pallas as pl
from jax.experimental.pallas import tpu as pltpu
from jax.experimental.pallas import tpu_sc as plsc

_B, _L, _D = 4096, 200, 64
_K = 5
_BT = 16
_NC, _NS = 2, 16
_NW = _NC * _NS
_CH = 128
_NB = 2
_ROWS = _CH * _NB

_NEG = np.int32(-2147483648)
_MASK = np.int32(-256)


_WC = float((1.0 - np.exp(-1.0)) / (1.0 - np.exp(-200.0)))
_WTAIL = float(_WC * np.exp(-float(_K)) * (1.0 - np.exp(-(float(_L) - _K)))
               / (1.0 - np.exp(-1.0)))


def _sc_gather(reco_p, search_p, user_table, idx_r, idx_s1, idx_s2, idx_u, nb):
    n_hist = nb * _L
    rows_per_w = n_hist // _NW
    chunks_per_w = rows_per_w // _ROWS
    idxrows_per_w = rows_per_w // _CH
    u_per_w = nb // _NW

    mesh = plsc.VectorSubcoreMesh(core_axis_name="c", subcore_axis_name="s")

    @functools.partial(
        pl.kernel,
        mesh=mesh,
        out_type=[
            jax.ShapeDtypeStruct((n_hist, 128), jnp.float32),
            jax.ShapeDtypeStruct((n_hist, 128), jnp.float32),
            jax.ShapeDtypeStruct((n_hist, 128), jnp.float32),
            jax.ShapeDtypeStruct((nb, _D), jnp.float32),
        ],
        scratch_types=[
            pltpu.VMEM((2, _NB, _CH), jnp.int32),
            pltpu.VMEM((2, _ROWS, 128), jnp.float32),
            pltpu.VMEM((1, _CH), jnp.int32),
            pltpu.VMEM((u_per_w, _D), jnp.float32),
            pltpu.SemaphoreType.DMA,
            pltpu.SemaphoreType.DMA,
        ],
    )
    def k(rt, st, ut, ir, is1, is2, iu, g_r, g_s1, g_s2, g_u,
          idx_v, rows_v, uidx_v, urow_v, sem0, sem1):
        wid = lax.axis_index("s") * _NC + lax.axis_index("c")
        sems = (sem0, sem1)

        def stream(tbl, idx_hbm, out_hbm):
            base = wid * idxrows_per_w

            def fire(c, b):
                pltpu.sync_copy(idx_hbm.at[pl.ds(base + c * _NB, _NB)],
                                idx_v.at[b])
                for j in range(_NB):
                    pltpu.async_copy(tbl.at[idx_v.at[b, j]],
                                     rows_v.at[b, pl.ds(j * _CH, _CH)],
                                     sems[b])

            def drain(b):
                for j in range(_NB):
                    pltpu.make_async_copy(tbl.at[idx_v.at[b, j]],
                                          rows_v.at[b, pl.ds(j * _CH, _CH)],
                                          sems[b]).wait()

            fire(0, 0)

            def body(cc, carry):
                for b in range(2):
                    c = 2 * cc + b

                    @pl.when(c + 1 < chunks_per_w)
                    def _():
                        fire(c + 1, 1 - b)

                    drain(b)
                    pltpu.sync_copy(
                        rows_v.at[b],
                        out_hbm.at[pl.ds((base + c * _NB) * _CH, _ROWS)])
                return carry

            lax.fori_loop(0, chunks_per_w // 2, body, 0)

        stream(rt, ir, g_r)
        stream(st, is1, g_s1)
        stream(st, is2, g_s2)

        div = _CH // u_per_w
        pltpu.sync_copy(iu.at[pl.ds(wid // div, 1)], uidx_v)
        uoff = (wid % div) * u_per_w

        def ubody(rnd, carry):
            uvec = uidx_v[0, pl.ds(uoff + rnd * 16, 16)]
            descs = []
            for j in range(16):
                uid = uvec[j]
                descs.append(pltpu.async_copy(
                    ut.at[pl.ds(uid, 1)],
                    urow_v.at[pl.ds(rnd * 16 + j, 1)],
                    sem0,
                ))
            for d in descs:
                d.wait()
            return carry

        lax.fori_loop(0, u_per_w // 16, ubody, 0)
        pltpu.sync_copy(urow_v, g_u.at[pl.ds(wid * u_per_w, u_per_w)])

    return k(reco_p, search_p, user_table, idx_r, idx_s1, idx_s2, idx_u)


def _owa_block(x):
    bt = x.shape[0]
    i32 = lax.bitcast_convert_type(x, jnp.int32)
    s = jnp.where(i32 >= 0, i32, _NEG - i32)
    pos = lax.broadcasted_iota(jnp.int32, x.shape, 1)
    key = (s & _MASK) | pos

    def val(m):
        sq = m & _MASK
        iq = jnp.where(sq >= 0, sq, _NEG - sq)
        return lax.bitcast_convert_type(iq, jnp.float32)

    def body(kk, carry):
        key, acc, m_prev = carry
        key = jnp.where(key == m_prev, _NEG, key)
        m = jnp.max(key, axis=1, keepdims=True)
        wk = jnp.float32(_WC) * jnp.exp(-kk.astype(jnp.float32))
        acc = acc + val(m) * wk
        return key, acc, m

    neg0 = jnp.full((bt, 1, 128), _NEG, jnp.int32)
    key, acc, m = lax.fori_loop(
        0, _K, body, (key, jnp.zeros((bt, 1, 128), jnp.float32), neg0))
    key = jnp.where(key == m, _NEG, key)
    m = jnp.max(key, axis=1, keepdims=True)
    acc = acc + val(m) * jnp.float32(_WTAIL)
    return acc.reshape(bt, 128)


def _head_body(g0a, g0b, g1a, g1b, g2a, g2b, u, t,
               w1a, w1b, w1c, w1d, w1e, b1, w2, b2, out_ref):
    def owa2(ga, gb):
        xa = ga[...].reshape(_BT, _L, 128)[:, :, :_D]
        xb = gb[...].reshape(_BT, _L, 128)[:, :, :_D]
        acc = _owa_block(jnp.concatenate([xa, xb], axis=2))
        return jnp.concatenate([acc[:, :_D], acc[:, _D:]], axis=0)

    x0 = owa2(g0a, g0b)
    x1 = owa2(g1a, g1b)
    x2 = owa2(g2a, g2b)
    h = (
        jnp.dot(x0, w1a[...], preferred_element_type=jnp.float32)
        + jnp.dot(x1, w1b[...], preferred_element_type=jnp.float32)
        + jnp.dot(x2, w1c[...], preferred_element_type=jnp.float32)
        + jnp.dot(u[...], w1d[...], preferred_element_type=jnp.float32)
        + jnp.dot(t[...], w1e[...], preferred_element_type=jnp.float32)
        + b1[...]
    )
    h = jnp.where(h >= 0, h, h * jnp.float32(0.01))
    out_ref[...] = jnp.dot(h, w2[...], preferred_element_type=jnp.float32) + b2[...]


def _tc_head(G0, G1, G2, U, T, w1a, w1b, w1c, w1d, w1e, b1, W2, b2, nb):
    bt2 = 2 * _BT
    grid = nb // bt2
    blk = _BT * _L
    biga = lambda: pl.BlockSpec((blk, 128), lambda i: (2 * i, 0))
    bigb = lambda: pl.BlockSpec((blk, 128), lambda i: (2 * i + 1, 0))
    return pl.pallas_call(
        _head_body,
        grid=(grid,),
        in_specs=[
            biga(), bigb(), biga(), bigb(), biga(), bigb(),
            pl.BlockSpec((bt2, _D), lambda i: (i, 0)),
            pl.BlockSpec((bt2, 6), lambda i: (i, 0)),
            pl.BlockSpec((_D, _D), lambda i: (0, 0)),
            pl.BlockSpec((_D, _D), lambda i: (0, 0)),
            pl.BlockSpec((_D, _D), lambda i: (0, 0)),
            pl.BlockSpec((_D, _D), lambda i: (0, 0)),
            pl.BlockSpec((6, _D), lambda i: (0, 0)),
            pl.BlockSpec((1, _D), lambda i: (0, 0)),
            pl.BlockSpec((_D, 2), lambda i: (0, 0)),
            pl.BlockSpec((1, 2), lambda i: (0, 0)),
        ],
        out_specs=pl.BlockSpec((bt2, 2), lambda i: (i, 0)),
        out_shape=jax.ShapeDtypeStruct((nb, 2), jnp.float32),
    )(G0, G0, G1, G1, G2, G2, U, T, w1a, w1b, w1c, w1d, w1e, b1, W2, b2)


_NSPLIT = 2


def kernel(reco_history, search_history, open_search_history, time_features, user_id,
           reco_table, search_table, user_table, W1, b1, W2, b2):
    reco_p = jnp.pad(reco_table, ((0, 0), (0, 128 - _D)))
    search_p = jnp.pad(search_table, ((0, 0), (0, 128 - _D)))

    w1a = W1[0:_D]
    w1b = W1[_D:2 * _D]
    w1c = W1[2 * _D:3 * _D]
    w1d = W1[3 * _D:4 * _D]
    w1e = W1[4 * _D:]
    b1r = b1.reshape(1, _D)
    b2r = b2.reshape(1, 2)

    nb = _B // _NSPLIT
    n_hist = nb * _L
    outs = []
    for h in range(_NSPLIT):
        sl = slice(h * nb, (h + 1) * nb)
        idx_r = reco_history[sl].astype(jnp.int32).reshape(n_hist // _CH, _CH)
        idx_s1 = search_history[sl].astype(jnp.int32).reshape(n_hist // _CH, _CH)
        idx_s2 = open_search_history[sl].astype(jnp.int32).reshape(n_hist // _CH, _CH)
        idx_u = user_id[sl].astype(jnp.int32).reshape(nb // _CH, _CH)

        g_r, g_s1, g_s2, g_u = _sc_gather(
            reco_p, search_p, user_table, idx_r, idx_s1, idx_s2, idx_u, nb)
        outs.append(_tc_head(g_r, g_s1, g_s2, g_u, time_features[sl],
                             w1a, w1b, w1c, w1d, w1e, b1r, W2, b2r, nb))
    return jnp.concatenate(outs, axis=0)


# speedup vs baseline: 1.0865x; 1.0865x over previous
import functools

import jax
import jax.numpy as jnp
import numpy as np
from jax import lax
from jax.experimental import pallas as pl
from jax.experimental.pallas import tpu as pltpu
from jax.experimental.pallas import tpu_sc as plsc

_B, _L, _D = 4096, 200, 64
_K = 5
_BT = 16
_NC, _NS = 2, 16
_NW = _NC * _NS
_CH = 128
_NB = 1
_ROWS = _CH * _NB

_NEG = np.int32(-2147483648)
_MASK = np.int32(-256)


_WC = float((1.0 - np.exp(-1.0)) / (1.0 - np.exp(-200.0)))
_WTAIL = float(_WC * np.exp(-float(_K)) * (1.0 - np.exp(-(float(_L) - _K)))
               / (1.0 - np.exp(-1.0)))


def _sc_gather(reco_p, search_p, user_table, idx_r, idx_s1, idx_s2, idx_u, nb):
    n_hist = nb * _L
    rows_per_w = n_hist // _NW
    chunks_per_w = rows_per_w // _ROWS
    idxrows_per_w = rows_per_w // _CH
    u_per_w = nb // _NW

    mesh = plsc.VectorSubcoreMesh(core_axis_name="c", subcore_axis_name="s")

    @functools.partial(
        pl.kernel,
        mesh=mesh,
        out_type=[
            jax.ShapeDtypeStruct((n_hist, 128), jnp.float32),
            jax.ShapeDtypeStruct((n_hist, 128), jnp.float32),
            jax.ShapeDtypeStruct((n_hist, 128), jnp.float32),
            jax.ShapeDtypeStruct((nb, _D), jnp.float32),
        ],
        scratch_types=[
            pltpu.VMEM((2, _NB, _CH), jnp.int32),
            pltpu.VMEM((2, _ROWS, 128), jnp.float32),
            pltpu.VMEM((1, _CH), jnp.int32),
            pltpu.VMEM((u_per_w, _D), jnp.float32),
            pltpu.SemaphoreType.DMA,
            pltpu.SemaphoreType.DMA,
        ],
    )
    def k(rt, st, ut, ir, is1, is2, iu, g_r, g_s1, g_s2, g_u,
          idx_v, rows_v, uidx_v, urow_v, sem0, sem1):
        wid = lax.axis_index("s") * _NC + lax.axis_index("c")
        sems = (sem0, sem1)

        def stream(tbl, idx_hbm, out_hbm):
            base = wid * idxrows_per_w

            def fire(c, b):
                pltpu.sync_copy(idx_hbm.at[pl.ds(base + c * _NB, _NB)],
                                idx_v.at[b])
                for j in range(_NB):
                    pltpu.async_copy(tbl.at[idx_v.at[b, j]],
                                     rows_v.at[b, pl.ds(j * _CH, _CH)],
                                     sems[b])

            def drain(b):
                for j in range(_NB):
                    pltpu.make_async_copy(tbl.at[idx_v.at[b, j]],
                                          rows_v.at[b, pl.ds(j * _CH, _CH)],
                                          sems[b]).wait()

            fire(0, 0)

            def body(cc, carry):
                for b in range(2):
                    c = 2 * cc + b

                    @pl.when(c + 1 < chunks_per_w)
                    def _():
                        fire(c + 1, 1 - b)

                    drain(b)
                    pltpu.sync_copy(
                        rows_v.at[b],
                        out_hbm.at[pl.ds((base + c * _NB) * _CH, _ROWS)])
                return carry

            lax.fori_loop(0, chunks_per_w // 2, body, 0)

        stream(rt, ir, g_r)
        stream(st, is1, g_s1)
        stream(st, is2, g_s2)

        div = _CH // u_per_w
        pltpu.sync_copy(iu.at[pl.ds(wid // div, 1)], uidx_v)
        uoff = (wid % div) * u_per_w

        def ubody(rnd, carry):
            uvec = uidx_v[0, pl.ds(uoff + rnd * 16, 16)]
            descs = []
            for j in range(16):
                uid = uvec[j]
                descs.append(pltpu.async_copy(
                    ut.at[pl.ds(uid, 1)],
                    urow_v.at[pl.ds(rnd * 16 + j, 1)],
                    sem0,
                ))
            for d in descs:
                d.wait()
            return carry

        lax.fori_loop(0, u_per_w // 16, ubody, 0)
        pltpu.sync_copy(urow_v, g_u.at[pl.ds(wid * u_per_w, u_per_w)])

    return k(reco_p, search_p, user_table, idx_r, idx_s1, idx_s2, idx_u)


def _owa_block(x):
    bt = x.shape[0]
    i32 = lax.bitcast_convert_type(x, jnp.int32)
    s = jnp.where(i32 >= 0, i32, _NEG - i32)
    pos = lax.broadcasted_iota(jnp.int32, x.shape, 1)
    key = (s & _MASK) | pos

    def val(m):
        sq = m & _MASK
        iq = jnp.where(sq >= 0, sq, _NEG - sq)
        return lax.bitcast_convert_type(iq, jnp.float32)

    def body(kk, carry):
        key, acc, m_prev = carry
        key = jnp.where(key == m_prev, _NEG, key)
        m = jnp.max(key, axis=1, keepdims=True)
        wk = jnp.float32(_WC) * jnp.exp(-kk.astype(jnp.float32))
        acc = acc + val(m) * wk
        return key, acc, m

    neg0 = jnp.full((bt, 1, 128), _NEG, jnp.int32)
    key, acc, m = lax.fori_loop(
        0, _K, body, (key, jnp.zeros((bt, 1, 128), jnp.float32), neg0))
    key = jnp.where(key == m, _NEG, key)
    m = jnp.max(key, axis=1, keepdims=True)
    acc = acc + val(m) * jnp.float32(_WTAIL)
    return acc.reshape(bt, 128)


def _head_body(g0a, g0b, g1a, g1b, g2a, g2b, u, t,
               w1a, w1b, w1c, w1d, w1e, b1, w2, b2, out_ref):
    def owa2(ga, gb):
        xa = ga[...].reshape(_BT, _L, 128)[:, :, :_D]
        xb = gb[...].reshape(_BT, _L, 128)[:, :, :_D]
        acc = _owa_block(jnp.concatenate([xa, xb], axis=2))
        return jnp.concatenate([acc[:, :_D], acc[:, _D:]], axis=0)

    x0 = owa2(g0a, g0b)
    x1 = owa2(g1a, g1b)
    x2 = owa2(g2a, g2b)
    h = (
        jnp.dot(x0, w1a[...], preferred_element_type=jnp.float32)
        + jnp.dot(x1, w1b[...], preferred_element_type=jnp.float32)
        + jnp.dot(x2, w1c[...], preferred_element_type=jnp.float32)
        + jnp.dot(u[...], w1d[...], preferred_element_type=jnp.float32)
        + jnp.dot(t[...], w1e[...], preferred_element_type=jnp.float32)
        + b1[...]
    )
    h = jnp.where(h >= 0, h, h * jnp.float32(0.01))
    out_ref[...] = jnp.dot(h, w2[...], preferred_element_type=jnp.float32) + b2[...]


def _tc_head(G0, G1, G2, U, T, w1a, w1b, w1c, w1d, w1e, b1, W2, b2, nb):
    bt2 = 2 * _BT
    grid = nb // bt2
    blk = _BT * _L
    biga = lambda: pl.BlockSpec((blk, 128), lambda i: (2 * i, 0))
    bigb = lambda: pl.BlockSpec((blk, 128), lambda i: (2 * i + 1, 0))
    return pl.pallas_call(
        _head_body,
        grid=(grid,),
        in_specs=[
            biga(), bigb(), biga(), bigb(), biga(), bigb(),
            pl.BlockSpec((bt2, _D), lambda i: (i, 0)),
            pl.BlockSpec((bt2, 6), lambda i: (i, 0)),
            pl.BlockSpec((_D, _D), lambda i: (0, 0)),
            pl.BlockSpec((_D, _D), lambda i: (0, 0)),
            pl.BlockSpec((_D, _D), lambda i: (0, 0)),
            pl.BlockSpec((_D, _D), lambda i: (0, 0)),
            pl.BlockSpec((6, _D), lambda i: (0, 0)),
            pl.BlockSpec((1, _D), lambda i: (0, 0)),
            pl.BlockSpec((_D, 2), lambda i: (0, 0)),
            pl.BlockSpec((1, 2), lambda i: (0, 0)),
        ],
        out_specs=pl.BlockSpec((bt2, 2), lambda i: (i, 0)),
        out_shape=jax.ShapeDtypeStruct((nb, 2), jnp.float32),
    )(G0, G0, G1, G1, G2, G2, U, T, w1a, w1b, w1c, w1d, w1e, b1, W2, b2)


_NSPLIT = 4


def kernel(reco_history, search_history, open_search_history, time_features, user_id,
           reco_table, search_table, user_table, W1, b1, W2, b2):
    reco_p = jnp.pad(reco_table, ((0, 0), (0, 128 - _D)))
    search_p = jnp.pad(search_table, ((0, 0), (0, 128 - _D)))

    w1a = W1[0:_D]
    w1b = W1[_D:2 * _D]
    w1c = W1[2 * _D:3 * _D]
    w1d = W1[3 * _D:4 * _D]
    w1e = W1[4 * _D:]
    b1r = b1.reshape(1, _D)
    b2r = b2.reshape(1, 2)

    nb = _B // _NSPLIT
    n_hist = nb * _L
    outs = []
    for h in range(_NSPLIT):
        sl = slice(h * nb, (h + 1) * nb)
        idx_r = reco_history[sl].astype(jnp.int32).reshape(n_hist // _CH, _CH)
        idx_s1 = search_history[sl].astype(jnp.int32).reshape(n_hist // _CH, _CH)
        idx_s2 = open_search_history[sl].astype(jnp.int32).reshape(n_hist // _CH, _CH)
        idx_u = user_id[sl].astype(jnp.int32).reshape(nb // _CH, _CH)

        g_r, g_s1, g_s2, g_u = _sc_gather(
            reco_p, search_p, user_table, idx_r, idx_s1, idx_s2, idx_u, nb)
        outs.append(_tc_head(g_r, g_s1, g_s2, g_u, time_features[sl],
                             w1a, w1b, w1c, w1d, w1e, b1r, W2, b2r, nb))
    return jnp.concatenate(outs, axis=0)
